# baseline (speedup 1.0000x reference)
import jax
import jax.numpy as jnp
from jax import lax
from jax.experimental import pallas as pl
from jax.experimental.pallas import tpu as pltpu
from jax.experimental.pallas import tpu_sc as plsc


def _pos_embed_body(rw_hbm, cw_hbm, out_hbm, rw_v, cw_v, slab_v, sem):
    nc = 2
    wid = lax.axis_index("s") * nc + lax.axis_index("c")

    pltpu.sync_copy(rw_hbm, rw_v)
    pltpu.sync_copy(cw_hbm, cw_v)

    iota16 = lax.iota(jnp.int32, 16)

    @pl.when(wid < 16)
    def _build_x():
        c0 = wid * 16
        rows = [rw_v[pl.ds(x * 256 + c0, 16)] for x in range(32)]
        t_lo, t_hi = [], []
        for j in range(16):
            lo = jnp.full((16,), rows[0][j], jnp.float32)
            hi = jnp.full((16,), rows[16][j], jnp.float32)
            for x in range(1, 16):
                lo = jnp.where(iota16 == x,
                               jnp.full((16,), rows[x][j], jnp.float32), lo)
                hi = jnp.where(iota16 == x,
                               jnp.full((16,), rows[x + 16][j], jnp.float32),
                               hi)
            t_lo.append(lo)
            t_hi.append(hi)

        def row(y, _):
            for j in range(16):
                off = j * 1024 + y * 32
                slab_v[pl.ds(off, 16)] = t_lo[j]
                slab_v[pl.ds(off + 16, 16)] = t_hi[j]
            return 0

        lax.fori_loop(0, 32, row, 0)

    @pl.when(wid >= 16)
    def _build_y():
        c0 = (wid - 16) * 16
        rows = [cw_v[pl.ds(y * 256 + c0, 16)] for y in range(32)]
        for y in range(32):
            for j in range(16):
                v = jnp.full((16,), rows[y][j], jnp.float32)
                off = j * 1024 + y * 32
                slab_v[pl.ds(off, 16)] = v
                slab_v[pl.ds(off + 16, 16)] = v

    handles = [
        pltpu.async_copy(slab_v, out_hbm.at[b, wid], sem) for b in range(16)
    ]
    for h in handles:
        h.wait()


def kernel(uv_feat, row_weight, col_weight):
    b = uv_feat.shape[0]
    h, w = uv_feat.shape[-2], uv_feat.shape[-1]
    d = row_weight.shape[-1]
    assert (b, h, w, d) == (16, 32, 32, 256)

    mesh = plsc.VectorSubcoreMesh(core_axis_name="c", subcore_axis_name="s")
    run = pl.kernel(
        _pos_embed_body,
        mesh=mesh,
        out_type=jax.ShapeDtypeStruct((b, 32, 16 * h * w), jnp.float32),
        scratch_types=[
            pltpu.VMEM((32 * d,), jnp.float32),
            pltpu.VMEM((32 * d,), jnp.float32),
            pltpu.VMEM((16 * h * w,), jnp.float32),
            pltpu.SemaphoreType.DMA,
        ],
    )
    out = run(
        row_weight[:w].reshape(w * d),
        col_weight[:h].reshape(h * d),
    )
    return out.reshape(b, 2 * d, h, w)
